# baseline (device time: 33441 ns/iter reference)
import jax
import jax.numpy as jnp
from jax import lax
from jax.experimental import pallas as pl
from jax.experimental.pallas import tpu as pltpu

QR = 512
CS = 128
DX = 176
DY = 168
DZ = 168


def kernel(x):
    m, n = x.shape

    def body(x_ref, out_ref, recv_x_ref, recv_y_ref, recv_z_ref,
             sx_send, sx_recv, sy_send, sy_recv, sz_send, sz_recv):
        my_x = lax.axis_index("x")
        my_y = lax.axis_index("y")
        my_z = lax.axis_index("z")
        qy = lax.rem(my_y, 2)
        qz = lax.rem(my_z, 2)
        partner = (1 - my_x, my_y, my_z)
        b_y = (my_x, my_y + 1 - 2 * qy, my_z)
        b_z = (my_x, my_y, my_z + 1 - 2 * qz)

        r_me = (2 * qy + qz) * QR
        r_y = (2 * (1 - qy) + qz) * QR
        r_z = (2 * qy + (1 - qz)) * QR
        r_d = (2 * (1 - qy) + (1 - qz)) * QR

        barrier = pltpu.get_barrier_semaphore()
        for nbr in (partner, b_y, b_z):
            pl.semaphore_signal(
                barrier, inc=1, device_id=nbr,
                device_id_type=pl.DeviceIdType.MESH,
            )
        pl.semaphore_wait(barrier, 3)

        def rcopy(src, dst, ssem, rsem, dev):
            return pltpu.make_async_remote_copy(
                src_ref=src, dst_ref=dst, send_sem=ssem, recv_sem=rsem,
                device_id=dev, device_id_type=pl.DeviceIdType.MESH,
            )

        order = (2, 1, 3, 0)
        rx = [None] * 4
        for k in order:
            rx[k] = rcopy(x_ref.at[pl.ds(r_me + k * CS, CS), :],
                          recv_x_ref.at[pl.ds(k * CS, CS), :],
                          sx_send.at[k], sx_recv.at[k], partner)
            rx[k].start()
        rx4 = rcopy(x_ref.at[pl.ds(r_d, DX), :],
                    recv_x_ref.at[pl.ds(QR, DX), :],
                    sx_send.at[4], sx_recv.at[4], partner)
        rx4.start()

        ry, rz = [None] * 4, [None] * 4

        def fwd(k):
            rx[k].wait_recv()
            ry[k] = rcopy(recv_x_ref.at[pl.ds(k * CS, CS), :],
                          recv_y_ref.at[pl.ds(k * CS, CS), :],
                          sy_send.at[k], sy_recv.at[k], b_y)
            ry[k].start()
            rz[k] = rcopy(recv_x_ref.at[pl.ds(k * CS, CS), :],
                          recv_z_ref.at[pl.ds(k * CS, CS), :],
                          sz_send.at[k], sz_recv.at[k], b_z)
            rz[k].start()

        fwd(2)
        fwd(1)
        rz[1].wait_recv()
        rz[2].wait_recv()
        ry4 = rcopy(recv_z_ref.at[pl.ds(DX, DY), :],
                    recv_y_ref.at[pl.ds(QR, DY), :],
                    sy_send.at[4], sy_recv.at[4], b_y)
        ry4.start()
        fwd(3)
        fwd(0)
        ry[2].wait_recv()
        ry[3].wait_recv()
        rz4 = rcopy(recv_y_ref.at[pl.ds(DX + DY, DZ), :],
                    recv_z_ref.at[pl.ds(QR, DZ), :],
                    sz_send.at[4], sz_recv.at[4], b_z)
        rz4.start()

        sl = pl.ds(r_me, QR)
        out_ref[sl, :] = x_ref[sl, :] + recv_x_ref[pl.ds(0, QR), :]

        ry[0].wait_recv()
        ry[1].wait_recv()
        sl = pl.ds(r_y, QR)
        out_ref[sl, :] = x_ref[sl, :] + recv_y_ref[pl.ds(0, QR), :]

        rz[0].wait_recv()
        rz[3].wait_recv()
        sl = pl.ds(r_z, QR)
        out_ref[sl, :] = x_ref[sl, :] + recv_z_ref[pl.ds(0, QR), :]

        rx4.wait_recv()
        sl = pl.ds(r_d, DX)
        out_ref[sl, :] = x_ref[sl, :] + recv_x_ref[pl.ds(QR, DX), :]

        ry4.wait_recv()
        sl = pl.ds(r_d + DX, DY)
        out_ref[sl, :] = x_ref[sl, :] + recv_y_ref[pl.ds(QR, DY), :]

        rz4.wait_recv()
        sl = pl.ds(r_d + DX + DY, DZ)
        out_ref[sl, :] = x_ref[sl, :] + recv_z_ref[pl.ds(QR, DZ), :]

        for r in rx + ry + rz + [rx4, ry4, rz4]:
            r.wait_send()

    return pl.pallas_call(
        body,
        out_shape=jax.ShapeDtypeStruct((m, n), x.dtype),
        in_specs=[pl.BlockSpec(memory_space=pltpu.VMEM)],
        out_specs=pl.BlockSpec(memory_space=pltpu.VMEM),
        scratch_shapes=[
            pltpu.VMEM((QR + DX, n), x.dtype),
            pltpu.VMEM((QR + DY, n), x.dtype),
            pltpu.VMEM((QR + DZ, n), x.dtype),
            pltpu.SemaphoreType.DMA((5,)),
            pltpu.SemaphoreType.DMA((5,)),
            pltpu.SemaphoreType.DMA((5,)),
            pltpu.SemaphoreType.DMA((5,)),
            pltpu.SemaphoreType.DMA((5,)),
            pltpu.SemaphoreType.DMA((5,)),
        ],
        compiler_params=pltpu.CompilerParams(collective_id=0),
    )(x)


# device time: 28043 ns/iter; 1.1925x vs baseline; 1.1925x over previous
import jax
import jax.numpy as jnp
from jax import lax
from jax.experimental import pallas as pl
from jax.experimental.pallas import tpu as pltpu

QR = 512
CS = 128
DX = 176
DY = 168
DZ = 168


def kernel(x):
    m, n = x.shape

    def body(x_ref, out_ref, recv_x_ref, recv_y_ref, recv_z_ref,
             sx_send, sx_recv, sy_send, sy_recv, sz_send, sz_recv):
        my_x = lax.axis_index("x")
        my_y = lax.axis_index("y")
        my_z = lax.axis_index("z")
        qy = lax.rem(my_y, 2)
        qz = lax.rem(my_z, 2)
        partner = (1 - my_x, my_y, my_z)
        b_y = (my_x, my_y + 1 - 2 * qy, my_z)
        b_z = (my_x, my_y, my_z + 1 - 2 * qz)

        r_me = (2 * qy + qz) * QR
        r_y = (2 * (1 - qy) + qz) * QR
        r_z = (2 * qy + (1 - qz)) * QR
        r_d = (2 * (1 - qy) + (1 - qz)) * QR

        barrier = pltpu.get_barrier_semaphore()
        for nbr in (partner, b_y, b_z):
            pl.semaphore_signal(
                barrier, inc=1, device_id=nbr,
                device_id_type=pl.DeviceIdType.MESH,
            )
        pl.semaphore_wait(barrier, 3)

        def rcopy(src, dst, ssem, rsem, dev):
            return pltpu.make_async_remote_copy(
                src_ref=src, dst_ref=dst, send_sem=ssem, recv_sem=rsem,
                device_id=dev, device_id_type=pl.DeviceIdType.MESH,
            )

        order = (2, 1, 3, 0)
        rx = [None] * 4
        for k in order:
            rx[k] = rcopy(x_ref.at[pl.ds(r_me + k * CS, CS), :],
                          recv_x_ref.at[pl.ds(k * CS, CS), :],
                          sx_send.at[k], sx_recv.at[k], partner)
            rx[k].start()
        rx4 = rcopy(x_ref.at[pl.ds(r_d, DX), :],
                    recv_x_ref.at[pl.ds(QR, DX), :],
                    sx_send.at[4], sx_recv.at[4], partner)
        rx4.start()

        ry, rz = [None] * 4, [None] * 4

        def fwd(k):
            rx[k].wait_recv()
            ry[k] = rcopy(recv_x_ref.at[pl.ds(k * CS, CS), :],
                          recv_y_ref.at[pl.ds(k * CS, CS), :],
                          sy_send.at[k], sy_recv.at[k], b_y)
            ry[k].start()
            rz[k] = rcopy(recv_x_ref.at[pl.ds(k * CS, CS), :],
                          recv_z_ref.at[pl.ds(k * CS, CS), :],
                          sz_send.at[k], sz_recv.at[k], b_z)
            rz[k].start()

        for k in order:
            fwd(k)
        rz[1].wait_recv()
        rz[2].wait_recv()
        ry4 = rcopy(recv_z_ref.at[pl.ds(DX, DY), :],
                    recv_y_ref.at[pl.ds(QR, DY), :],
                    sy_send.at[4], sy_recv.at[4], b_y)
        ry4.start()
        ry[2].wait_recv()
        ry[3].wait_recv()
        rz4 = rcopy(recv_y_ref.at[pl.ds(DX + DY, DZ), :],
                    recv_z_ref.at[pl.ds(QR, DZ), :],
                    sz_send.at[4], sz_recv.at[4], b_z)
        rz4.start()

        sl = pl.ds(r_me, QR)
        out_ref[sl, :] = x_ref[sl, :] + recv_x_ref[pl.ds(0, QR), :]

        ry[0].wait_recv()
        ry[1].wait_recv()
        sl = pl.ds(r_y, QR)
        out_ref[sl, :] = x_ref[sl, :] + recv_y_ref[pl.ds(0, QR), :]

        rz[0].wait_recv()
        rz[3].wait_recv()
        sl = pl.ds(r_z, QR)
        out_ref[sl, :] = x_ref[sl, :] + recv_z_ref[pl.ds(0, QR), :]

        rx4.wait_recv()
        sl = pl.ds(r_d, DX)
        out_ref[sl, :] = x_ref[sl, :] + recv_x_ref[pl.ds(QR, DX), :]

        ry4.wait_recv()
        sl = pl.ds(r_d + DX, DY)
        out_ref[sl, :] = x_ref[sl, :] + recv_y_ref[pl.ds(QR, DY), :]

        rz4.wait_recv()
        sl = pl.ds(r_d + DX + DY, DZ)
        out_ref[sl, :] = x_ref[sl, :] + recv_z_ref[pl.ds(QR, DZ), :]

        for r in rx + ry + rz + [rx4, ry4, rz4]:
            r.wait_send()

    return pl.pallas_call(
        body,
        out_shape=jax.ShapeDtypeStruct((m, n), x.dtype),
        in_specs=[pl.BlockSpec(memory_space=pltpu.VMEM)],
        out_specs=pl.BlockSpec(memory_space=pltpu.VMEM),
        scratch_shapes=[
            pltpu.VMEM((QR + DX, n), x.dtype),
            pltpu.VMEM((QR + DY, n), x.dtype),
            pltpu.VMEM((QR + DZ, n), x.dtype),
            pltpu.SemaphoreType.DMA((5,)),
            pltpu.SemaphoreType.DMA((5,)),
            pltpu.SemaphoreType.DMA((5,)),
            pltpu.SemaphoreType.DMA((5,)),
            pltpu.SemaphoreType.DMA((5,)),
            pltpu.SemaphoreType.DMA((5,)),
        ],
        compiler_params=pltpu.CompilerParams(collective_id=0),
    )(x)


# device time: 4465 ns/iter; 7.4896x vs baseline; 6.2806x over previous
import jax
import jax.numpy as jnp
from jax import lax
from jax.experimental import pallas as pl
from jax.experimental.pallas import tpu as pltpu

QR = 512
CS = 64
NC = QR // CS
DX = 176
DY = 168
DZ = 168
ORDER = (2, 3, 4, 5, 6, 7, 1, 0)


def kernel(x):
    m, n = x.shape

    def body(x_ref, out_ref, recv_x_ref, recv_y_ref, recv_z_ref,
             sx_send, sx_recv, sy_send, sy_recv, sz_send, sz_recv):
        my_x = lax.axis_index("x")
        my_y = lax.axis_index("y")
        my_z = lax.axis_index("z")
        qy = lax.rem(my_y, 2)
        qz = lax.rem(my_z, 2)
        partner = (1 - my_x, my_y, my_z)
        b_y = (my_x, my_y + 1 - 2 * qy, my_z)
        b_z = (my_x, my_y, my_z + 1 - 2 * qz)

        r_me = (2 * qy + qz) * QR
        r_y = (2 * (1 - qy) + qz) * QR
        r_z = (2 * qy + (1 - qz)) * QR
        r_d = (2 * (1 - qy) + (1 - qz)) * QR

        barrier = pltpu.get_barrier_semaphore()
        for nbr in (partner, b_y, b_z):
            pl.semaphore_signal(
                barrier, inc=1, device_id=nbr,
                device_id_type=pl.DeviceIdType.MESH,
            )
        pl.semaphore_wait(barrier, 3)

        def rcopy(src, dst, ssem, rsem, dev):
            return pltpu.make_async_remote_copy(
                src_ref=src, dst_ref=dst, send_sem=ssem, recv_sem=rsem,
                device_id=dev, device_id_type=pl.DeviceIdType.MESH,
            )

        rx = [None] * NC
        for k in ORDER:
            rx[k] = rcopy(x_ref.at[pl.ds(r_me + k * CS, CS), :],
                          recv_x_ref.at[pl.ds(k * CS, CS), :],
                          sx_send.at[k], sx_recv.at[k], partner)
            rx[k].start()
        rx4 = rcopy(x_ref.at[pl.ds(r_d, DX), :],
                    recv_x_ref.at[pl.ds(QR, DX), :],
                    sx_send.at[NC], sx_recv.at[NC], partner)
        rx4.start()

        ry, rz = [None] * NC, [None] * NC
        for k in ORDER:
            rx[k].wait_recv()
            ry[k] = rcopy(recv_x_ref.at[pl.ds(k * CS, CS), :],
                          recv_y_ref.at[pl.ds(k * CS, CS), :],
                          sy_send.at[k], sy_recv.at[k], b_y)
            ry[k].start()
            rz[k] = rcopy(recv_x_ref.at[pl.ds(k * CS, CS), :],
                          recv_z_ref.at[pl.ds(k * CS, CS), :],
                          sz_send.at[k], sz_recv.at[k], b_z)
            rz[k].start()

        for k in (2, 3, 4, 5):
            rz[k].wait_recv()
        ry4 = rcopy(recv_z_ref.at[pl.ds(DX, DY), :],
                    recv_y_ref.at[pl.ds(QR, DY), :],
                    sy_send.at[NC], sy_recv.at[NC], b_y)
        ry4.start()
        for k in (6, 7):
            ry[k].wait_recv()
        rz4 = rcopy(recv_y_ref.at[pl.ds(DX + DY, DZ), :],
                    recv_z_ref.at[pl.ds(QR, DZ), :],
                    sz_send.at[NC], sz_recv.at[NC], b_z)
        rz4.start()

        sl = pl.ds(r_me, QR)
        out_ref[sl, :] = x_ref[sl, :] + recv_x_ref[pl.ds(0, QR), :]

        for k in (0, 1, 2, 3, 4, 5):
            ry[k].wait_recv()
        sl = pl.ds(r_y, QR)
        out_ref[sl, :] = x_ref[sl, :] + recv_y_ref[pl.ds(0, QR), :]

        for k in (0, 1, 6, 7):
            rz[k].wait_recv()
        sl = pl.ds(r_z, QR)
        out_ref[sl, :] = x_ref[sl, :] + recv_z_ref[pl.ds(0, QR), :]

        rx4.wait_recv()
        sl = pl.ds(r_d, DX)
        out_ref[sl, :] = x_ref[sl, :] + recv_x_ref[pl.ds(QR, DX), :]

        ry4.wait_recv()
        sl = pl.ds(r_d + DX, DY)
        out_ref[sl, :] = x_ref[sl, :] + recv_y_ref[pl.ds(QR, DY), :]

        rz4.wait_recv()
        sl = pl.ds(r_d + DX + DY, DZ)
        out_ref[sl, :] = x_ref[sl, :] + recv_z_ref[pl.ds(QR, DZ), :]

        for r in rx + ry + rz + [rx4, ry4, rz4]:
            r.wait_send()

    return pl.pallas_call(
        body,
        out_shape=jax.ShapeDtypeStruct((m, n), x.dtype),
        in_specs=[pl.BlockSpec(memory_space=pltpu.VMEM)],
        out_specs=pl.BlockSpec(memory_space=pltpu.VMEM),
        scratch_shapes=[
            pltpu.VMEM((QR + DX, n), x.dtype),
            pltpu.VMEM((QR + DY, n), x.dtype),
            pltpu.VMEM((QR + DZ, n), x.dtype),
            pltpu.SemaphoreType.DMA((NC + 1,)),
            pltpu.SemaphoreType.DMA((NC + 1,)),
            pltpu.SemaphoreType.DMA((NC + 1,)),
            pltpu.SemaphoreType.DMA((NC + 1,)),
            pltpu.SemaphoreType.DMA((NC + 1,)),
            pltpu.SemaphoreType.DMA((NC + 1,)),
        ],
        compiler_params=pltpu.CompilerParams(collective_id=0),
    )(x)
